# Initial kernel scaffold; baseline (speedup 1.0000x reference)
import jax
import jax.numpy as jnp
from jax.experimental import pallas as pl


def kernel(audio, label, text_proto):
    raise NotImplementedError("write your pallas kernel here")


# speedup vs baseline: 2.5979x; 2.5979x over previous
import functools

import jax
import jax.numpy as jnp
from jax import lax
from jax.experimental import pallas as pl
from jax.experimental.pallas import tpu as pltpu
from jax.experimental.pallas import tpu_sc as plsc

N_CLS = 1000
D = 512
B = 16384

NC = 2
NS = 16
L = 16
DH = D // NC
RPT = B // NS
CHUNK = 128
NCHUNK = RPT // CHUNK
CPAD = 1024
CPT = CPAD // NS
REAL_LAST = N_CLS - (NS - 1) * CPT


def _sc_body(audio, label, text, out,
             buf0, buf1, labels_v, ones_v, recip_v, cnt_v, sums_v, out_v,
             acc_sh, cnt_sh, sem0, sem1):
    c = lax.axis_index("c")
    s = lax.axis_index("s")
    row0 = s * RPT
    col0 = c * DH
    zeros16 = jnp.zeros((L,), jnp.float32)
    ones16 = jnp.ones((L,), jnp.float32)

    def zero_row(r, _):
        for v in range(DH // L):
            out_v[r, pl.ds(v * L, L)] = zeros16
        return 0
    lax.fori_loop(0, CPT, zero_row, 0)

    def fill_ones(r, _):
        ones_v[r, pl.ds(0, L)] = ones16
        return 0
    lax.fori_loop(0, CHUNK, fill_ones, 0)

    pltpu.sync_copy(out_v, acc_sh.at[pl.ds(s * CPT, CPT), :])
    pltpu.sync_copy(out_v.at[:, pl.ds(0, L)], cnt_sh.at[pl.ds(s * CPT, CPT), :])

    for j in range(NCHUNK):
        pltpu.sync_copy(label.at[pl.ds(row0 + j * CHUNK, CHUNK)], labels_v.at[j])

    plsc.subcore_barrier()

    bufs = (buf0, buf1)
    sems = (sem0, sem1)
    cps = [None, None]
    cps[0] = pltpu.async_copy(
        audio.at[pl.ds(row0, CHUNK), pl.ds(col0, DH)], bufs[0], sems[0])
    for j in range(NCHUNK):
        cps[j % 2].wait()
        if j + 1 < NCHUNK:
            cps[(j + 1) % 2] = pltpu.async_copy(
                audio.at[pl.ds(row0 + (j + 1) * CHUNK, CHUNK), pl.ds(col0, DH)],
                bufs[(j + 1) % 2], sems[(j + 1) % 2])
        idx = labels_v.at[j]
        pltpu.sync_copy(bufs[j % 2], acc_sh.at[idx], add=True)
        pltpu.sync_copy(ones_v, cnt_sh.at[idx], add=True)

    plsc.subcore_barrier()

    k0 = s * CPT
    pltpu.sync_copy(acc_sh.at[pl.ds(k0, CPT), :], sums_v)
    pltpu.sync_copy(cnt_sh.at[pl.ds(k0, CPT), :], cnt_v)
    zi16 = jnp.zeros((L,), jnp.int32)
    for v in range(CPT // L):
        rows = jax.lax.iota(jnp.int32, L) + v * L
        cnt = plsc.load_gather(cnt_v, [rows, zi16])
        rec = jnp.where(cnt > 0.0, 1.0 / jnp.maximum(cnt, 1.0), 0.0)
        recip_v[pl.ds(v * L, L)] = rec

    @pl.when(s < NS - 1)
    def _():
        pltpu.sync_copy(text.at[pl.ds(k0, CPT), pl.ds(col0, DH)], out_v)

    @pl.when(s == NS - 1)
    def _():
        pltpu.sync_copy(text.at[pl.ds((NS - 1) * CPT, REAL_LAST), pl.ds(col0, DH)],
                        out_v.at[pl.ds(0, REAL_LAST), :])

    def out_row(r, _):
        ridx = jnp.full((L,), r, jnp.int32)
        rec = plsc.load_gather(recip_v, [ridx])
        for v in range(DH // L):
            sl = pl.ds(v * L, L)
            out_v[r, sl] = out_v[r, sl] + sums_v[r, sl] * rec
        return 0
    lax.fori_loop(0, CPT, out_row, 0)

    @pl.when(s < NS - 1)
    def _():
        pltpu.sync_copy(out_v, out.at[pl.ds(k0, CPT), pl.ds(col0, DH)])

    @pl.when(s == NS - 1)
    def _():
        pltpu.sync_copy(out_v.at[pl.ds(0, REAL_LAST), :],
                        out.at[pl.ds((NS - 1) * CPT, REAL_LAST), pl.ds(col0, DH)])


@jax.jit
def kernel(audio, label, text_proto):
    mesh = plsc.VectorSubcoreMesh(core_axis_name="c", subcore_axis_name="s")
    run = pl.kernel(
        _sc_body,
        out_type=jax.ShapeDtypeStruct((N_CLS, D), jnp.float32),
        mesh=mesh,
        scratch_types=[
            pltpu.VMEM((CHUNK, DH), jnp.float32),
            pltpu.VMEM((CHUNK, DH), jnp.float32),
            pltpu.VMEM((NCHUNK, CHUNK), jnp.int32),
            pltpu.VMEM((CHUNK, L), jnp.float32),
            pltpu.VMEM((CPT,), jnp.float32),
            pltpu.VMEM((CPT, L), jnp.float32),
            pltpu.VMEM((CPT, DH), jnp.float32),
            pltpu.VMEM((CPT, DH), jnp.float32),
            pltpu.VMEM_SHARED((CPAD, DH), jnp.float32),
            pltpu.VMEM_SHARED((CPAD, L), jnp.float32),
            pltpu.SemaphoreType.DMA,
            pltpu.SemaphoreType.DMA,
        ],
        compiler_params=pltpu.CompilerParams(
            use_tc_tiling_on_sc=False, needs_layout_passes=False),
        name="proto_evolution_sc",
    )
    return run(audio, label, text_proto)


# speedup vs baseline: 3.9432x; 1.5178x over previous
import jax
import jax.numpy as jnp
from jax import lax
from jax.experimental import pallas as pl
from jax.experimental.pallas import tpu as pltpu
from jax.experimental.pallas import tpu_sc as plsc

N_CLS = 1000
D = 512
B = 16384

NC = 2
NS = 16
L = 16
TW = 128
RPT = B // NS
CHUNK = 128
NCHUNK = RPT // CHUNK
CPAD = 1024
CPT = CPAD // NS
REAL_LAST = N_CLS - (NS - 1) * CPT


def _sc_body(audio, label, text, out,
             bufA0, bufA1, bufB0, bufB1, labels_v, cnt1_v, cnt16_v, recip_v,
             sums0_v, sums1_v, out0_v, out1_v,
             acc0_sh, acc1_sh, cnt_sh, semA, semB, semC):
    c = lax.axis_index("c")
    s = lax.axis_index("s")
    row0 = s * RPT
    colA = c * (2 * TW)
    colB = colA + TW
    zeros16 = jnp.zeros((L,), jnp.float32)
    ones16 = jnp.ones((L,), jnp.float32)

    def zero_row(r, _):
        for v in range(TW // L):
            sl = pl.ds(v * L, L)
            out0_v[r, sl] = zeros16
            out1_v[r, sl] = zeros16
        return 0
    lax.fori_loop(0, CPT, zero_row, 0)

    def zero_cnt(i, _):
        cnt1_v[pl.ds(i * L, L)] = zeros16
        return 0
    lax.fori_loop(0, CPAD // L, zero_cnt, 0)

    pltpu.sync_copy(out0_v, acc0_sh.at[pl.ds(s * CPT, CPT), :])
    pltpu.sync_copy(out1_v, acc1_sh.at[pl.ds(s * CPT, CPT), :])

    for j in range(NCHUNK):
        pltpu.sync_copy(label.at[pl.ds(row0 + j * CHUNK, CHUNK)], labels_v.at[j])

    for j in range(NCHUNK):
        for v in range(CHUNK // L):
            lbl = labels_v[j, pl.ds(v * L, L)]
            plsc.addupdate_scatter(cnt1_v, [lbl], ones16)

    cws = []
    for t in range(NS):
        cws.append(pltpu.async_copy(
            cnt1_v.at[pl.ds(t * CPT, CPT)],
            cnt_sh.at[pl.ds(t * CPAD + s * CPT, CPT)], semC))
    for cw in cws:
        cw.wait()

    plsc.subcore_barrier()

    bufsA = (bufA0, bufA1)
    bufsB = (bufB0, bufB1)
    cpsA = [None, None]
    cpsB = [None, None]

    def start(j):
        rows = pl.ds(row0 + j * CHUNK, CHUNK)
        cpsA[j % 2] = pltpu.async_copy(
            audio.at[rows, pl.ds(colA, TW)], bufsA[j % 2], semA)
        cpsB[j % 2] = pltpu.async_copy(
            audio.at[rows, pl.ds(colB, TW)], bufsB[j % 2], semB)

    start(0)
    for j in range(NCHUNK):
        cpsA[j % 2].wait()
        cpsB[j % 2].wait()
        if j + 1 < NCHUNK:
            start(j + 1)
        idx = labels_v.at[j]
        pltpu.sync_copy(bufsA[j % 2], acc0_sh.at[idx], add=True)
        pltpu.sync_copy(bufsB[j % 2], acc1_sh.at[idx], add=True)

    plsc.subcore_barrier()

    k0 = s * CPT
    pltpu.sync_copy(acc0_sh.at[pl.ds(k0, CPT), :], sums0_v)
    pltpu.sync_copy(acc1_sh.at[pl.ds(k0, CPT), :], sums1_v)
    pltpu.sync_copy(cnt_sh.at[pl.ds(s * CPAD, CPAD)], cnt16_v)

    for v in range(CPT // L):
        a = zeros16
        for r in range(NS):
            a = a + cnt16_v[pl.ds(r * CPT + v * L, L)]
        recip_v[pl.ds(v * L, L)] = jnp.where(
            a > 0.0, 1.0 / jnp.maximum(a, 1.0), 0.0)

    @pl.when(s < NS - 1)
    def _():
        pltpu.sync_copy(text.at[pl.ds(k0, CPT), pl.ds(colA, TW)], out0_v)
        pltpu.sync_copy(text.at[pl.ds(k0, CPT), pl.ds(colB, TW)], out1_v)

    @pl.when(s == NS - 1)
    def _():
        rows = pl.ds((NS - 1) * CPT, REAL_LAST)
        pltpu.sync_copy(text.at[rows, pl.ds(colA, TW)],
                        out0_v.at[pl.ds(0, REAL_LAST), :])
        pltpu.sync_copy(text.at[rows, pl.ds(colB, TW)],
                        out1_v.at[pl.ds(0, REAL_LAST), :])

    def out_row(r, _):
        ridx = jnp.full((L,), r, jnp.int32)
        rec = plsc.load_gather(recip_v, [ridx])
        for v in range(TW // L):
            sl = pl.ds(v * L, L)
            out0_v[r, sl] = out0_v[r, sl] + sums0_v[r, sl] * rec
            out1_v[r, sl] = out1_v[r, sl] + sums1_v[r, sl] * rec
        return 0
    lax.fori_loop(0, CPT, out_row, 0)

    @pl.when(s < NS - 1)
    def _():
        pltpu.sync_copy(out0_v, out.at[pl.ds(k0, CPT), pl.ds(colA, TW)])
        pltpu.sync_copy(out1_v, out.at[pl.ds(k0, CPT), pl.ds(colB, TW)])

    @pl.when(s == NS - 1)
    def _():
        rows = pl.ds((NS - 1) * CPT, REAL_LAST)
        pltpu.sync_copy(out0_v.at[pl.ds(0, REAL_LAST), :],
                        out.at[rows, pl.ds(colA, TW)])
        pltpu.sync_copy(out1_v.at[pl.ds(0, REAL_LAST), :],
                        out.at[rows, pl.ds(colB, TW)])


@jax.jit
def kernel(audio, label, text_proto):
    mesh = plsc.VectorSubcoreMesh(core_axis_name="c", subcore_axis_name="s")
    run = pl.kernel(
        _sc_body,
        out_type=jax.ShapeDtypeStruct((N_CLS, D), jnp.float32),
        mesh=mesh,
        scratch_types=[
            pltpu.VMEM((CHUNK, TW), jnp.float32),
            pltpu.VMEM((CHUNK, TW), jnp.float32),
            pltpu.VMEM((CHUNK, TW), jnp.float32),
            pltpu.VMEM((CHUNK, TW), jnp.float32),
            pltpu.VMEM((NCHUNK, CHUNK), jnp.int32),
            pltpu.VMEM((CPAD,), jnp.float32),
            pltpu.VMEM((CPAD,), jnp.float32),
            pltpu.VMEM((CPT,), jnp.float32),
            pltpu.VMEM((CPT, TW), jnp.float32),
            pltpu.VMEM((CPT, TW), jnp.float32),
            pltpu.VMEM((CPT, TW), jnp.float32),
            pltpu.VMEM((CPT, TW), jnp.float32),
            pltpu.VMEM_SHARED((CPAD, TW), jnp.float32),
            pltpu.VMEM_SHARED((CPAD, TW), jnp.float32),
            pltpu.VMEM_SHARED((NS * CPAD,), jnp.float32),
            pltpu.SemaphoreType.DMA,
            pltpu.SemaphoreType.DMA,
            pltpu.SemaphoreType.DMA,
        ],
        compiler_params=pltpu.CompilerParams(
            use_tc_tiling_on_sc=True, needs_layout_passes=False),
        name="proto_evolution_sc",
    )
    return run(audio, label, text_proto)


# speedup vs baseline: 4.3243x; 1.0967x over previous
import jax
import jax.numpy as jnp
from jax import lax
from jax.experimental import pallas as pl
from jax.experimental.pallas import tpu as pltpu
from jax.experimental.pallas import tpu_sc as plsc

N_CLS = 1000
D = 512
B = 16384

NC = 2
NS = 16
L = 16
TW = 128
RPT = B // NS
CHUNK = 128
NCHUNK = RPT // CHUNK
NBUF = 3
CPAD = 1024
CPT = CPAD // NS
REAL_LAST = N_CLS - (NS - 1) * CPT


def _sc_body(audio, label, text, out,
             bufA0, bufA1, bufA2, bufB0, bufB1, bufB2,
             labels_v, cnt1_v, cnt16_v, recip_v,
             acc0_sh, acc1_sh, cnt_sh, semA, semB, semS, semC):
    c = lax.axis_index("c")
    s = lax.axis_index("s")
    row0 = s * RPT
    colA = c * (2 * TW)
    colB = colA + TW
    zeros16 = jnp.zeros((L,), jnp.float32)
    ones16 = jnp.ones((L,), jnp.float32)

    bufsA = (bufA0, bufA1, bufA2)
    bufsB = (bufB0, bufB1, bufB2)
    gA = [None] * NCHUNK
    gB = [None] * NCHUNK
    sA = [None] * NCHUNK
    sB = [None] * NCHUNK

    def start_gather(j):
        rows = pl.ds(row0 + j * CHUNK, CHUNK)
        gA[j] = pltpu.async_copy(
            audio.at[rows, pl.ds(colA, TW)], bufsA[j % NBUF], semA)
        gB[j] = pltpu.async_copy(
            audio.at[rows, pl.ds(colB, TW)], bufsB[j % NBUF], semB)

    for j in range(min(NBUF, NCHUNK)):
        start_gather(j)

    def zero_row(r, _):
        for v in range(TW // L):
            bufB0[r, pl.ds(v * L, L)] = zeros16
        return 0
    lax.fori_loop(0, CPT, zero_row, 0)

    def zero_cnt(i, _):
        cnt1_v[pl.ds(i * L, L)] = zeros16
        return 0
    lax.fori_loop(0, CPAD // L, zero_cnt, 0)

    zsrc = bufB0.at[pl.ds(0, CPT), :]
    pltpu.sync_copy(zsrc, acc0_sh.at[pl.ds(s * CPT, CPT), :])
    pltpu.sync_copy(zsrc, acc1_sh.at[pl.ds(s * CPT, CPT), :])

    for j in range(NCHUNK):
        pltpu.sync_copy(label.at[pl.ds(row0 + j * CHUNK, CHUNK)], labels_v.at[j])

    for j in range(NCHUNK):
        for v in range(CHUNK // L):
            lbl = labels_v[j, pl.ds(v * L, L)]
            plsc.addupdate_scatter(cnt1_v, [lbl], ones16)

    cws = []
    for t in range(NS):
        cws.append(pltpu.async_copy(
            cnt1_v.at[pl.ds(t * CPT, CPT)],
            cnt_sh.at[pl.ds(t * CPAD + s * CPT, CPT)], semC))
    for cw in cws:
        cw.wait()

    plsc.subcore_barrier()

    waited = [False] * NCHUNK
    for j in range(NCHUNK):
        if 1 <= j and j + 2 < NCHUNK:
            sA[j - 1].wait()
            sB[j - 1].wait()
            waited[j - 1] = True
            start_gather(j + 2)
        gA[j].wait()
        gB[j].wait()
        idx = labels_v.at[j]
        sA[j] = pltpu.async_copy(bufsA[j % NBUF], acc0_sh.at[idx], semS, add=True)
        sB[j] = pltpu.async_copy(bufsB[j % NBUF], acc1_sh.at[idx], semS, add=True)
    for j in range(NCHUNK):
        if not waited[j]:
            sA[j].wait()
            sB[j].wait()

    plsc.subcore_barrier()

    k0 = s * CPT
    sums0_v = bufA0
    sums1_v = bufA1
    out0_v = bufB0
    out1_v = bufB1
    f0 = pltpu.async_copy(acc0_sh.at[pl.ds(k0, CPT), :],
                          sums0_v.at[pl.ds(0, CPT), :], semA)
    f1 = pltpu.async_copy(acc1_sh.at[pl.ds(k0, CPT), :],
                          sums1_v.at[pl.ds(0, CPT), :], semB)
    f2 = pltpu.async_copy(cnt_sh.at[pl.ds(s * CPAD, CPAD)], cnt16_v, semC)

    @pl.when(s < NS - 1)
    def _():
        pltpu.sync_copy(text.at[pl.ds(k0, CPT), pl.ds(colA, TW)],
                        out0_v.at[pl.ds(0, CPT), :])
        pltpu.sync_copy(text.at[pl.ds(k0, CPT), pl.ds(colB, TW)],
                        out1_v.at[pl.ds(0, CPT), :])

    @pl.when(s == NS - 1)
    def _():
        rows = pl.ds((NS - 1) * CPT, REAL_LAST)
        pltpu.sync_copy(text.at[rows, pl.ds(colA, TW)],
                        out0_v.at[pl.ds(0, REAL_LAST), :])
        pltpu.sync_copy(text.at[rows, pl.ds(colB, TW)],
                        out1_v.at[pl.ds(0, REAL_LAST), :])

    f2.wait()
    for v in range(CPT // L):
        a = zeros16
        for r in range(NS):
            a = a + cnt16_v[pl.ds(r * CPT + v * L, L)]
        recip_v[pl.ds(v * L, L)] = jnp.where(
            a > 0.0, 1.0 / jnp.maximum(a, 1.0), 0.0)
    f0.wait()
    f1.wait()

    def out_row(r, _):
        ridx = jnp.full((L,), r, jnp.int32)
        rec = plsc.load_gather(recip_v, [ridx])
        for v in range(TW // L):
            sl = pl.ds(v * L, L)
            out0_v[r, sl] = out0_v[r, sl] + sums0_v[r, sl] * rec
            out1_v[r, sl] = out1_v[r, sl] + sums1_v[r, sl] * rec
        return 0
    lax.fori_loop(0, CPT, out_row, 0)

    @pl.when(s < NS - 1)
    def _():
        pltpu.sync_copy(out0_v.at[pl.ds(0, CPT), :],
                        out.at[pl.ds(k0, CPT), pl.ds(colA, TW)])
        pltpu.sync_copy(out1_v.at[pl.ds(0, CPT), :],
                        out.at[pl.ds(k0, CPT), pl.ds(colB, TW)])

    @pl.when(s == NS - 1)
    def _():
        rows = pl.ds((NS - 1) * CPT, REAL_LAST)
        pltpu.sync_copy(out0_v.at[pl.ds(0, REAL_LAST), :],
                        out.at[rows, pl.ds(colA, TW)])
        pltpu.sync_copy(out1_v.at[pl.ds(0, REAL_LAST), :],
                        out.at[rows, pl.ds(colB, TW)])


@jax.jit
def kernel(audio, label, text_proto):
    mesh = plsc.VectorSubcoreMesh(core_axis_name="c", subcore_axis_name="s")
    run = pl.kernel(
        _sc_body,
        out_type=jax.ShapeDtypeStruct((N_CLS, D), jnp.float32),
        mesh=mesh,
        scratch_types=[
            pltpu.VMEM((CHUNK, TW), jnp.float32),
            pltpu.VMEM((CHUNK, TW), jnp.float32),
            pltpu.VMEM((CHUNK, TW), jnp.float32),
            pltpu.VMEM((CHUNK, TW), jnp.float32),
            pltpu.VMEM((CHUNK, TW), jnp.float32),
            pltpu.VMEM((CHUNK, TW), jnp.float32),
            pltpu.VMEM((NCHUNK, CHUNK), jnp.int32),
            pltpu.VMEM((CPAD,), jnp.float32),
            pltpu.VMEM((CPAD,), jnp.float32),
            pltpu.VMEM((CPT,), jnp.float32),
            pltpu.VMEM_SHARED((CPAD, TW), jnp.float32),
            pltpu.VMEM_SHARED((CPAD, TW), jnp.float32),
            pltpu.VMEM_SHARED((NS * CPAD,), jnp.float32),
            pltpu.SemaphoreType.DMA,
            pltpu.SemaphoreType.DMA,
            pltpu.SemaphoreType.DMA,
            pltpu.SemaphoreType.DMA,
        ],
        compiler_params=pltpu.CompilerParams(
            use_tc_tiling_on_sc=True, needs_layout_passes=False),
        name="proto_evolution_sc",
    )
    return run(audio, label, text_proto)


# speedup vs baseline: 6.5602x; 1.5170x over previous
import jax
import jax.numpy as jnp
from jax import lax
from jax.experimental import pallas as pl
from jax.experimental.pallas import tpu as pltpu
from jax.experimental.pallas import tpu_sc as plsc

N_CLS = 1000
D = 512
B = 16384

NC = 2
NS = 16
L = 16
TW = 128
RPT = B // NS
CHUNK = 128
NCHUNK = RPT // CHUNK
NBUF = 3
CPAD = 1024
CPT = CPAD // NS
REAL_LAST = N_CLS - (NS - 1) * CPT


def _sc_body(audio, label, text, out,
             bufA0, bufA1, bufA2, bufB0, bufB1, bufB2,
             labels_v, cnt1_v, cnt16_v, recip_v,
             acc0_sh, acc1_sh, cnt_sh, semA, semB, semS, semC):
    c = lax.axis_index("c")
    s = lax.axis_index("s")
    row0 = s * RPT
    colA = c * (2 * TW)
    colB = colA + TW
    zeros16 = jnp.zeros((L,), jnp.float32)
    ones16 = jnp.ones((L,), jnp.float32)

    bufsA = (bufA0, bufA1, bufA2)
    bufsB = (bufB0, bufB1, bufB2)
    gA = [None] * NCHUNK
    gB = [None] * NCHUNK
    sA = [None] * NCHUNK
    sB = [None] * NCHUNK

    def start_gather(j):
        rows = pl.ds(row0 + j * CHUNK, CHUNK)
        gA[j] = pltpu.async_copy(
            audio.at[rows, pl.ds(colA, TW)], bufsA[j % NBUF], semA)
        gB[j] = pltpu.async_copy(
            audio.at[rows, pl.ds(colB, TW)], bufsB[j % NBUF], semB)

    start_gather(0)
    start_gather(1)

    def zero_row(r, _):
        for v in range(TW // L):
            bufB2[r, pl.ds(v * L, L)] = zeros16
        return 0
    lax.fori_loop(0, CPT, zero_row, 0)

    def zero_cnt(i, _):
        cnt1_v[pl.ds(i * L, L)] = zeros16
        return 0
    lax.fori_loop(0, CPAD // L, zero_cnt, 0)

    zsrc = bufB2.at[pl.ds(0, CPT), :]
    pltpu.sync_copy(zsrc, acc0_sh.at[pl.ds(s * CPT, CPT), :])
    pltpu.sync_copy(zsrc, acc1_sh.at[pl.ds(s * CPT, CPT), :])

    for j in range(NCHUNK):
        pltpu.sync_copy(label.at[pl.ds(row0 + j * CHUNK, CHUNK)], labels_v.at[j])

    for j in range(NCHUNK):
        for v in range(CHUNK // L):
            lbl = labels_v[j, pl.ds(v * L, L)]
            plsc.addupdate_scatter(cnt1_v, [lbl], ones16)

    cws = []
    for t in range(NS):
        cws.append(pltpu.async_copy(
            cnt1_v.at[pl.ds(t * CPT, CPT)],
            cnt_sh.at[pl.ds(t * CPAD + s * CPT, CPT)], semC))
    for cw in cws:
        cw.wait()

    plsc.subcore_barrier()

    waited = [False] * NCHUNK
    for j in range(NCHUNK):
        if j + 2 < NCHUNK:
            if j >= 1:
                sA[j - 1].wait()
                sB[j - 1].wait()
                waited[j - 1] = True
            start_gather(j + 2)
        gA[j].wait()
        gB[j].wait()
        idx = labels_v.at[j]
        sA[j] = pltpu.async_copy(bufsA[j % NBUF], acc0_sh.at[idx], semS, add=True)
        sB[j] = pltpu.async_copy(bufsB[j % NBUF], acc1_sh.at[idx], semS, add=True)
    for j in range(NCHUNK):
        if not waited[j]:
            sA[j].wait()
            sB[j].wait()

    plsc.subcore_barrier()

    k0 = s * CPT
    sums0_v = bufA0
    sums1_v = bufA1
    out0_v = bufB0
    out1_v = bufB1
    f0 = pltpu.async_copy(acc0_sh.at[pl.ds(k0, CPT), :],
                          sums0_v.at[pl.ds(0, CPT), :], semA)
    f1 = pltpu.async_copy(acc1_sh.at[pl.ds(k0, CPT), :],
                          sums1_v.at[pl.ds(0, CPT), :], semB)
    f2 = pltpu.async_copy(cnt_sh.at[pl.ds(s * CPAD, CPAD)], cnt16_v, semC)

    @pl.when(s < NS - 1)
    def _():
        pltpu.sync_copy(text.at[pl.ds(k0, CPT), pl.ds(colA, TW)],
                        out0_v.at[pl.ds(0, CPT), :])
        pltpu.sync_copy(text.at[pl.ds(k0, CPT), pl.ds(colB, TW)],
                        out1_v.at[pl.ds(0, CPT), :])

    @pl.when(s == NS - 1)
    def _():
        rows = pl.ds((NS - 1) * CPT, REAL_LAST)
        pltpu.sync_copy(text.at[rows, pl.ds(colA, TW)],
                        out0_v.at[pl.ds(0, REAL_LAST), :])
        pltpu.sync_copy(text.at[rows, pl.ds(colB, TW)],
                        out1_v.at[pl.ds(0, REAL_LAST), :])

    f2.wait()
    for v in range(CPT // L):
        a = zeros16
        for r in range(NS):
            a = a + cnt16_v[pl.ds(r * CPT + v * L, L)]
        recip_v[pl.ds(v * L, L)] = jnp.where(
            a > 0.0, 1.0 / jnp.maximum(a, 1.0), 0.0)
    f0.wait()
    f1.wait()

    def out_row(r, _):
        ridx = jnp.full((L,), r, jnp.int32)
        rec = plsc.load_gather(recip_v, [ridx])
        for v in range(TW // L):
            sl = pl.ds(v * L, L)
            out0_v[r, sl] = out0_v[r, sl] + sums0_v[r, sl] * rec
            out1_v[r, sl] = out1_v[r, sl] + sums1_v[r, sl] * rec
        return 0
    lax.fori_loop(0, CPT, out_row, 0)

    @pl.when(s < NS - 1)
    def _():
        pltpu.sync_copy(out0_v.at[pl.ds(0, CPT), :],
                        out.at[pl.ds(k0, CPT), pl.ds(colA, TW)])
        pltpu.sync_copy(out1_v.at[pl.ds(0, CPT), :],
                        out.at[pl.ds(k0, CPT), pl.ds(colB, TW)])

    @pl.when(s == NS - 1)
    def _():
        rows = pl.ds((NS - 1) * CPT, REAL_LAST)
        pltpu.sync_copy(out0_v.at[pl.ds(0, REAL_LAST), :],
                        out.at[rows, pl.ds(colA, TW)])
        pltpu.sync_copy(out1_v.at[pl.ds(0, REAL_LAST), :],
                        out.at[rows, pl.ds(colB, TW)])


@jax.jit
def kernel(audio, label, text_proto):
    mesh = plsc.VectorSubcoreMesh(core_axis_name="c", subcore_axis_name="s")
    run = pl.kernel(
        _sc_body,
        out_type=jax.ShapeDtypeStruct((N_CLS, D), jnp.float32),
        mesh=mesh,
        scratch_types=[
            pltpu.VMEM((CHUNK, TW), jnp.float32),
            pltpu.VMEM((CHUNK, TW), jnp.float32),
            pltpu.VMEM((CHUNK, TW), jnp.float32),
            pltpu.VMEM((CHUNK, TW), jnp.float32),
            pltpu.VMEM((CHUNK, TW), jnp.float32),
            pltpu.VMEM((CHUNK, TW), jnp.float32),
            pltpu.VMEM((NCHUNK, CHUNK), jnp.int32),
            pltpu.VMEM((CPAD,), jnp.float32),
            pltpu.VMEM((CPAD,), jnp.float32),
            pltpu.VMEM((CPT,), jnp.float32),
            pltpu.VMEM_SHARED((CPAD, TW), jnp.float32),
            pltpu.VMEM_SHARED((CPAD, TW), jnp.float32),
            pltpu.VMEM_SHARED((NS * CPAD,), jnp.float32),
            pltpu.SemaphoreType.DMA,
            pltpu.SemaphoreType.DMA,
            pltpu.SemaphoreType.DMA,
            pltpu.SemaphoreType.DMA,
        ],
        compiler_params=pltpu.CompilerParams(
            use_tc_tiling_on_sc=True, needs_layout_passes=False),
        name="proto_evolution_sc",
    )
    return run(audio, label, text_proto)
